# baseline (speedup 1.0000x reference)
import jax
import jax.numpy as jnp
from jax import lax
from jax.experimental import pallas as pl
from jax.experimental.pallas import tpu as pltpu
from jax.experimental.pallas import tpu_sc as plsc

N = 10000
E = 320000
F_IN = 128
H = 256
HH = 128
C_OUT = 40
NT = 16
CH = 128
BI = 20
NBLK = 8
EPT = CH * BI * NBLK
EP = EPT * NT
NP = 10240
RPT = NP // NT
NCHT = EPT // CH
EPS = 1e-5


def _agg_body(h_hbm, src_hbm, dst_hbm, out_hbm,
              isrc0, isrc1, idst0, idst1, buf0, buf1, acc,
              g0, g1, g0b, g1b, p0, p1):
    c = lax.axis_index("c")
    s = lax.axis_index("s")

    zv = jnp.zeros((16,), jnp.float32)

    def _z(i, _):
        r = i // (HH // 16)
        k = lax.rem(i, HH // 16) * 16
        buf0[r, pl.ds(k, 16)] = zv
        return 0

    lax.fori_loop(0, CH * (HH // 16), _z, 0)
    for k in range(RPT // CH):
        pltpu.sync_copy(buf0, acc.at[pl.ds(s * RPT + k * CH, CH)])
    plsc.subcore_barrier()

    off = c * NP

    def _rebase(A):
        def _rb(i, _):
            r = i // (CH // 16)
            k = lax.rem(i, CH // 16) * 16
            A[r, pl.ds(k, 16)] = A[r, pl.ds(k, 16)] + off
            return 0
        lax.fori_loop(0, BI * (CH // 16), _rb, 0)

    def _do_block(A, B, A2, B2, i):
        if i > 0:
            pltpu.make_async_copy(src_hbm.at[s, i], A, p0).wait()
            pltpu.make_async_copy(dst_hbm.at[s, i], B, p1).wait()
            _rebase(A)
        if i + 1 < NBLK:
            pltpu.async_copy(src_hbm.at[s, i + 1], A2, p0)
            pltpu.async_copy(dst_hbm.at[s, i + 1], B2, p1)

        def _gather(j, bf, sa, sb):
            pltpu.async_copy(h_hbm.at[A.at[j, pl.ds(0, CH // 2)]],
                             bf.at[pl.ds(0, CH // 2)], sa)
            pltpu.async_copy(h_hbm.at[A.at[j, pl.ds(CH // 2, CH // 2)]],
                             bf.at[pl.ds(CH // 2, CH // 2)], sb)

        def _gwait(j, bf, sa, sb):
            pltpu.make_async_copy(h_hbm.at[A.at[j, pl.ds(0, CH // 2)]],
                                  bf.at[pl.ds(0, CH // 2)], sa).wait()
            pltpu.make_async_copy(h_hbm.at[A.at[j, pl.ds(CH // 2, CH // 2)]],
                                  bf.at[pl.ds(CH // 2, CH // 2)], sb).wait()

        _gather(0, buf0, g0, g0b)

        def _pair(t, _):
            j0 = 2 * t
            j1 = j0 + 1
            _gwait(j0, buf0, g0, g0b)
            _gather(j1, buf1, g1, g1b)
            pltpu.sync_copy(buf0, acc.at[B.at[j0]], add=True)
            _gwait(j1, buf1, g1, g1b)

            @pl.when(t + 1 < BI // 2)
            def _():
                _gather(j0 + 2, buf0, g0, g0b)

            pltpu.sync_copy(buf1, acc.at[B.at[j1]], add=True)
            return 0

        lax.fori_loop(0, BI // 2, _pair, 0)

    pltpu.sync_copy(src_hbm.at[s, 0], isrc0)
    pltpu.sync_copy(dst_hbm.at[s, 0], idst0)
    _rebase(isrc0)
    for i in range(NBLK):
        if i % 2 == 0:
            _do_block(isrc0, idst0, isrc1, idst1, i)
        else:
            _do_block(isrc1, idst1, isrc0, idst0, i)
    plsc.subcore_barrier()

    base = s * RPT
    for k in range(RPT // CH):
        pltpu.sync_copy(acc.at[pl.ds(base + k * CH, CH)],
                        out_hbm.at[pl.ds(c * NP + base + k * CH, CH)])


@jax.jit
def _agg_call(h_split, src_both, dst):
    mesh = plsc.VectorSubcoreMesh(core_axis_name="c", subcore_axis_name="s")
    return pl.kernel(
        _agg_body,
        out_type=jax.ShapeDtypeStruct((2 * NP, HH), jnp.float32),
        mesh=mesh,
        scratch_types=[
            pltpu.VMEM((BI, CH), jnp.int32),
            pltpu.VMEM((BI, CH), jnp.int32),
            pltpu.VMEM((BI, CH), jnp.int32),
            pltpu.VMEM((BI, CH), jnp.int32),
            pltpu.VMEM((CH, HH), jnp.float32),
            pltpu.VMEM((CH, HH), jnp.float32),
            pltpu.VMEM_SHARED((NP, HH), jnp.float32),
            pltpu.SemaphoreType.DMA,
            pltpu.SemaphoreType.DMA,
            pltpu.SemaphoreType.DMA,
            pltpu.SemaphoreType.DMA,
            pltpu.SemaphoreType.DMA,
            pltpu.SemaphoreType.DMA,
        ],
    )(h_split, src_both, dst)


def _cnt_body(dst_hbm, out_hbm, idx_d, ones_v, zb, cnt_acc):
    c = lax.axis_index("c")
    s = lax.axis_index("s")

    pltpu.sync_copy(dst_hbm.at[s], idx_d)

    ov = jnp.ones((16,), jnp.float32)
    for k in range(CH // 16):
        ones_v[pl.ds(k * 16, 16)] = ov

    zv = jnp.zeros((16,), jnp.float32)

    @pl.when(s == 0)
    def _():
        def _z(i, _):
            zb[pl.ds(i * 16, 16)] = zv
            return 0
        lax.fori_loop(0, NP // 16, _z, 0)
        pltpu.sync_copy(zb, cnt_acc)

    plsc.subcore_barrier()

    def _count(j, _):
        pltpu.sync_copy(ones_v, cnt_acc.at[idx_d.at[j]], add=True)
        return 0

    lax.fori_loop(0, NCHT, _count, 0)
    plsc.subcore_barrier()

    @pl.when(s == 0)
    def _():
        pltpu.sync_copy(cnt_acc.at[pl.ds(c * (N // 2), N // 2)],
                        zb.at[pl.ds(0, N // 2)])
        pltpu.sync_copy(zb.at[pl.ds(0, N // 2)],
                        out_hbm.at[pl.ds(c * (N // 2), N // 2)])


@jax.jit
def _cnt_call(dst):
    mesh = plsc.VectorSubcoreMesh(core_axis_name="c", subcore_axis_name="s")
    return pl.kernel(
        _cnt_body,
        out_type=jax.ShapeDtypeStruct((N,), jnp.float32),
        mesh=mesh,
        scratch_types=[
            pltpu.VMEM((NCHT, CH), jnp.int32),
            pltpu.VMEM((CH,), jnp.float32),
            pltpu.VMEM((NP,), jnp.float32),
            pltpu.VMEM_SHARED((NP,), jnp.float32),
        ],
    )(dst)


def _bn_relu(t, g, b):
    mu = jnp.mean(t, axis=0, keepdims=True)
    var = jnp.mean((t - mu) ** 2, axis=0, keepdims=True)
    y = (t - mu) * lax.rsqrt(var + EPS) * g + b
    return jnp.maximum(y, 0.0)


def _split_store(out_ref, y):
    out_ref[pl.ds(0, N), :] = y[:, :HH]
    out_ref[pl.ds(NP, N), :] = y[:, HH:]


def _pre_body(x_ref, w_ref, b_ref, g_ref, bb_ref, out_ref):
    h = jnp.dot(x_ref[...], w_ref[...], preferred_element_type=jnp.float32)
    y = _bn_relu(h + b_ref[...], g_ref[...], bb_ref[...])
    _split_store(out_ref, y)


def _sage_tail(ms_ref, cnt_ref, h_ref, wl_ref, bl_ref, wr_ref):
    inv = 1.0 / jnp.maximum(cnt_ref[...], 1.0)
    m0 = ms_ref[pl.ds(0, N), :] * inv
    m1 = ms_ref[pl.ds(NP, N), :] * inv
    h0 = h_ref[pl.ds(0, N), :]
    h1 = h_ref[pl.ds(NP, N), :]
    t = (jnp.dot(m0, wl_ref[pl.ds(0, HH), :], preferred_element_type=jnp.float32)
         + jnp.dot(m1, wl_ref[pl.ds(HH, HH), :], preferred_element_type=jnp.float32)
         + jnp.dot(h0, wr_ref[pl.ds(0, HH), :], preferred_element_type=jnp.float32)
         + jnp.dot(h1, wr_ref[pl.ds(HH, HH), :], preferred_element_type=jnp.float32))
    return t + bl_ref[...]


def _mid_body(ms_ref, cnt_ref, h_ref, wl_ref, bl_ref, wr_ref, g_ref, bb_ref,
              out_ref):
    t = _sage_tail(ms_ref, cnt_ref, h_ref, wl_ref, bl_ref, wr_ref)
    y = _bn_relu(t, g_ref[...], bb_ref[...])
    _split_store(out_ref, y)


def _post_body(ms_ref, cnt_ref, h_ref, wl_ref, bl_ref, wr_ref, g4_ref, b4_ref,
               wp1_ref, bp1_ref, g5_ref, b5_ref, wp2_ref, bp2_ref, out_ref):
    t = _sage_tail(ms_ref, cnt_ref, h_ref, wl_ref, bl_ref, wr_ref)
    y = _bn_relu(t, g4_ref[...], b4_ref[...])
    u = jnp.dot(y, wp1_ref[...], preferred_element_type=jnp.float32) + bp1_ref[...]
    y5 = _bn_relu(u, g5_ref[...], b5_ref[...])
    z = jnp.dot(y5, wp2_ref[...], preferred_element_type=jnp.float32) + bp2_ref[...]
    zm = jnp.max(z, axis=1, keepdims=True)
    lse = jnp.log(jnp.sum(jnp.exp(z - zm), axis=1, keepdims=True)) + zm
    out_ref[...] = z - lse


def _tc_call(body, out_shape, *args):
    return pl.pallas_call(
        body, out_shape=jax.ShapeDtypeStruct(out_shape, jnp.float32))(*args)


def _pad_edges(edge_index):
    npad = EP - E
    pad_src = jnp.arange(npad, dtype=jnp.int32) % N
    pad_dst = N + jnp.arange(npad, dtype=jnp.int32) % (NP - N)
    src = jnp.concatenate([edge_index[0], pad_src])
    dst = jnp.concatenate([edge_index[1], pad_dst])
    return src.reshape(NT, NBLK, BI, CH), dst.reshape(NT, NBLK, BI, CH)


def kernel(x, edge_index, W_pre, b_pre, bn1_g, bn1_b, W1l, b1l, W1r,
           bn2_g, bn2_b, W2l, b2l, W2r, bn3_g, bn3_b, W3l, b3l, W3r,
           bn4_g, bn4_b, W_post1, b_post1, bn5_g, bn5_b, W_post2, b_post2):
    src_both, dst = _pad_edges(edge_index)

    cnt = _cnt_call(dst.reshape(NT, NCHT, CH)).reshape(N, 1)

    r2 = lambda v: v.reshape(1, -1)
    h = _tc_call(_pre_body, (2 * NP, HH),
                 x, W_pre.T, r2(b_pre), r2(bn1_g), r2(bn1_b))

    ms = _agg_call(h, src_both, dst)
    h = _tc_call(_mid_body, (2 * NP, HH),
                 ms, cnt, h, W1l.T, r2(b1l), W1r.T, r2(bn2_g), r2(bn2_b))

    ms = _agg_call(h, src_both, dst)
    h = _tc_call(_mid_body, (2 * NP, HH),
                 ms, cnt, h, W2l.T, r2(b2l), W2r.T, r2(bn3_g), r2(bn3_b))

    ms = _agg_call(h, src_both, dst)
    out = _tc_call(_post_body, (N, C_OUT),
                   ms, cnt, h, W3l.T, r2(b3l), W3r.T, r2(bn4_g), r2(bn4_b),
                   W_post1.T, r2(b_post1), r2(bn5_g), r2(bn5_b),
                   W_post2.T, r2(b_post2))
    return out


# speedup vs baseline: 1.0139x; 1.0139x over previous
import jax
import jax.numpy as jnp
from jax import lax
from jax.experimental import pallas as pl
from jax.experimental.pallas import tpu as pltpu
from jax.experimental.pallas import tpu_sc as plsc

N = 10000
E = 320000
F_IN = 128
H = 256
HH = 128
C_OUT = 40
NT = 16
CH = 128
BI = 20
NBLK = 8
EPT = CH * BI * NBLK
EP = EPT * NT
NP = 10240
RPT = NP // NT
NCHT = EPT // CH
EPS = 1e-5


def _agg_body(h_hbm, src_hbm, dst_hbm, out_hbm,
              isrc0, isrc1, idst0, idst1, buf0, buf1, acc,
              g0, g1, g0b, g1b, p0, p1):
    c = lax.axis_index("c")
    s = lax.axis_index("s")

    zv = jnp.zeros((16,), jnp.float32)

    def _z(r, _):
        for k in range(HH // 16):
            buf0[r, pl.ds(k * 16, 16)] = zv
        return 0

    lax.fori_loop(0, CH, _z, 0)
    for k in range(RPT // CH):
        pltpu.sync_copy(buf0, acc.at[pl.ds(s * RPT + k * CH, CH)])
    plsc.subcore_barrier()

    off = c * NP

    def _rebase(A):
        def _rb(i, _):
            r = i // (CH // 16)
            k = lax.rem(i, CH // 16) * 16
            A[r, pl.ds(k, 16)] = A[r, pl.ds(k, 16)] + off
            return 0
        lax.fori_loop(0, BI * (CH // 16), _rb, 0)

    def _do_block(A, B, A2, B2, i):
        if i > 0:
            pltpu.make_async_copy(src_hbm.at[s, i], A, p0).wait()
            pltpu.make_async_copy(dst_hbm.at[s, i], B, p1).wait()
            _rebase(A)
        if i + 1 < NBLK:
            pltpu.async_copy(src_hbm.at[s, i + 1], A2, p0)
            pltpu.async_copy(dst_hbm.at[s, i + 1], B2, p1)

        def _gather(j, bf, sa, sb):
            pltpu.async_copy(h_hbm.at[A.at[j, pl.ds(0, CH // 2)]],
                             bf.at[pl.ds(0, CH // 2)], sa)
            pltpu.async_copy(h_hbm.at[A.at[j, pl.ds(CH // 2, CH // 2)]],
                             bf.at[pl.ds(CH // 2, CH // 2)], sb)

        def _gwait(j, bf, sa, sb):
            pltpu.make_async_copy(h_hbm.at[A.at[j, pl.ds(0, CH // 2)]],
                                  bf.at[pl.ds(0, CH // 2)], sa).wait()
            pltpu.make_async_copy(h_hbm.at[A.at[j, pl.ds(CH // 2, CH // 2)]],
                                  bf.at[pl.ds(CH // 2, CH // 2)], sb).wait()

        _gather(0, buf0, g0, g0b)

        def _pair(t, _):
            j0 = 2 * t
            j1 = j0 + 1
            _gwait(j0, buf0, g0, g0b)
            _gather(j1, buf1, g1, g1b)
            pltpu.sync_copy(buf0, acc.at[B.at[j0]], add=True)
            _gwait(j1, buf1, g1, g1b)

            @pl.when(t + 1 < BI // 2)
            def _():
                _gather(j0 + 2, buf0, g0, g0b)

            pltpu.sync_copy(buf1, acc.at[B.at[j1]], add=True)
            return 0

        lax.fori_loop(0, BI // 2, _pair, 0)

    pltpu.sync_copy(src_hbm.at[s, 0], isrc0)
    pltpu.sync_copy(dst_hbm.at[s, 0], idst0)
    _rebase(isrc0)
    for i in range(NBLK):
        if i % 2 == 0:
            _do_block(isrc0, idst0, isrc1, idst1, i)
        else:
            _do_block(isrc1, idst1, isrc0, idst0, i)
    plsc.subcore_barrier()

    base = s * RPT
    for k in range(RPT // CH):
        pltpu.sync_copy(acc.at[pl.ds(base + k * CH, CH)],
                        out_hbm.at[pl.ds(c * NP + base + k * CH, CH)])


@jax.jit
def _agg_call(h_split, src_both, dst):
    mesh = plsc.VectorSubcoreMesh(core_axis_name="c", subcore_axis_name="s")
    return pl.kernel(
        _agg_body,
        out_type=jax.ShapeDtypeStruct((2 * NP, HH), jnp.float32),
        mesh=mesh,
        scratch_types=[
            pltpu.VMEM((BI, CH), jnp.int32),
            pltpu.VMEM((BI, CH), jnp.int32),
            pltpu.VMEM((BI, CH), jnp.int32),
            pltpu.VMEM((BI, CH), jnp.int32),
            pltpu.VMEM((CH, HH), jnp.float32),
            pltpu.VMEM((CH, HH), jnp.float32),
            pltpu.VMEM_SHARED((NP, HH), jnp.float32),
            pltpu.SemaphoreType.DMA,
            pltpu.SemaphoreType.DMA,
            pltpu.SemaphoreType.DMA,
            pltpu.SemaphoreType.DMA,
            pltpu.SemaphoreType.DMA,
            pltpu.SemaphoreType.DMA,
        ],
    )(h_split, src_both, dst)


def _cnt_body(dst_hbm, out_hbm, idx_d, ones_v, zb, cnt_acc):
    c = lax.axis_index("c")
    s = lax.axis_index("s")

    pltpu.sync_copy(dst_hbm.at[s], idx_d)

    ov = jnp.ones((16,), jnp.float32)
    for k in range(CH // 16):
        ones_v[pl.ds(k * 16, 16)] = ov

    zv = jnp.zeros((16,), jnp.float32)

    @pl.when(s == 0)
    def _():
        def _z(i, _):
            zb[pl.ds(i * 16, 16)] = zv
            return 0
        lax.fori_loop(0, NP // 16, _z, 0)
        pltpu.sync_copy(zb, cnt_acc)

    plsc.subcore_barrier()

    def _count(j, _):
        pltpu.sync_copy(ones_v, cnt_acc.at[idx_d.at[j]], add=True)
        return 0

    lax.fori_loop(0, NCHT, _count, 0)
    plsc.subcore_barrier()

    @pl.when(s == 0)
    def _():
        pltpu.sync_copy(cnt_acc.at[pl.ds(c * (N // 2), N // 2)],
                        zb.at[pl.ds(0, N // 2)])
        pltpu.sync_copy(zb.at[pl.ds(0, N // 2)],
                        out_hbm.at[pl.ds(c * (N // 2), N // 2)])


@jax.jit
def _cnt_call(dst):
    mesh = plsc.VectorSubcoreMesh(core_axis_name="c", subcore_axis_name="s")
    return pl.kernel(
        _cnt_body,
        out_type=jax.ShapeDtypeStruct((N,), jnp.float32),
        mesh=mesh,
        scratch_types=[
            pltpu.VMEM((NCHT, CH), jnp.int32),
            pltpu.VMEM((CH,), jnp.float32),
            pltpu.VMEM((NP,), jnp.float32),
            pltpu.VMEM_SHARED((NP,), jnp.float32),
        ],
    )(dst)


def _bn_relu(t, g, b):
    mu = jnp.mean(t, axis=0, keepdims=True)
    var = jnp.mean((t - mu) ** 2, axis=0, keepdims=True)
    y = (t - mu) * lax.rsqrt(var + EPS) * g + b
    return jnp.maximum(y, 0.0)


def _split_store(out_ref, y):
    out_ref[pl.ds(0, N), :] = y[:, :HH]
    out_ref[pl.ds(NP, N), :] = y[:, HH:]


def _dot16(a, w):
    return jnp.dot(a.astype(jnp.bfloat16), w.astype(jnp.bfloat16),
                   preferred_element_type=jnp.float32)


def _pre_body(x_ref, w_ref, b_ref, g_ref, bb_ref, out_ref):
    h = _dot16(x_ref[...], w_ref[...])
    y = _bn_relu(h + b_ref[...], g_ref[...], bb_ref[...])
    _split_store(out_ref, y)


def _sage_tail(ms_ref, cnt_ref, h_ref, wl_ref, bl_ref, wr_ref):
    inv = 1.0 / jnp.maximum(cnt_ref[...], 1.0)
    m0 = ms_ref[pl.ds(0, N), :] * inv
    m1 = ms_ref[pl.ds(NP, N), :] * inv
    h0 = h_ref[pl.ds(0, N), :]
    h1 = h_ref[pl.ds(NP, N), :]
    t = (_dot16(m0, wl_ref[pl.ds(0, HH), :])
         + _dot16(m1, wl_ref[pl.ds(HH, HH), :])
         + _dot16(h0, wr_ref[pl.ds(0, HH), :])
         + _dot16(h1, wr_ref[pl.ds(HH, HH), :]))
    return t + bl_ref[...]


def _mid_body(ms_ref, cnt_ref, h_ref, wl_ref, bl_ref, wr_ref, g_ref, bb_ref,
              out_ref):
    t = _sage_tail(ms_ref, cnt_ref, h_ref, wl_ref, bl_ref, wr_ref)
    y = _bn_relu(t, g_ref[...], bb_ref[...])
    _split_store(out_ref, y)


def _post_body(ms_ref, cnt_ref, h_ref, wl_ref, bl_ref, wr_ref, g4_ref, b4_ref,
               wp1_ref, bp1_ref, g5_ref, b5_ref, wp2_ref, bp2_ref, out_ref):
    t = _sage_tail(ms_ref, cnt_ref, h_ref, wl_ref, bl_ref, wr_ref)
    y = _bn_relu(t, g4_ref[...], b4_ref[...])
    u = _dot16(y, wp1_ref[...]) + bp1_ref[...]
    y5 = _bn_relu(u, g5_ref[...], b5_ref[...])
    z = _dot16(y5, wp2_ref[...]) + bp2_ref[...]
    zm = jnp.max(z, axis=1, keepdims=True)
    lse = jnp.log(jnp.sum(jnp.exp(z - zm), axis=1, keepdims=True)) + zm
    out_ref[...] = z - lse


def _tc_call(body, out_shape, *args):
    return pl.pallas_call(
        body, out_shape=jax.ShapeDtypeStruct(out_shape, jnp.float32))(*args)


def _pad_edges(edge_index):
    npad = EP - E
    pad_src = jnp.arange(npad, dtype=jnp.int32) % N
    pad_dst = N + jnp.arange(npad, dtype=jnp.int32) % (NP - N)
    src = jnp.concatenate([edge_index[0], pad_src])
    dst = jnp.concatenate([edge_index[1], pad_dst])
    return src.reshape(NT, NBLK, BI, CH), dst.reshape(NT, NBLK, BI, CH)


def kernel(x, edge_index, W_pre, b_pre, bn1_g, bn1_b, W1l, b1l, W1r,
           bn2_g, bn2_b, W2l, b2l, W2r, bn3_g, bn3_b, W3l, b3l, W3r,
           bn4_g, bn4_b, W_post1, b_post1, bn5_g, bn5_b, W_post2, b_post2):
    src_both, dst = _pad_edges(edge_index)

    cnt = _cnt_call(dst.reshape(NT, NCHT, CH)).reshape(N, 1)

    r2 = lambda v: v.reshape(1, -1)
    h = _tc_call(_pre_body, (2 * NP, HH),
                 x, W_pre.T, r2(b_pre), r2(bn1_g), r2(bn1_b))

    ms = _agg_call(h, src_both, dst)
    h = _tc_call(_mid_body, (2 * NP, HH),
                 ms, cnt, h, W1l.T, r2(b1l), W1r.T, r2(bn2_g), r2(bn2_b))

    ms = _agg_call(h, src_both, dst)
    h = _tc_call(_mid_body, (2 * NP, HH),
                 ms, cnt, h, W2l.T, r2(b2l), W2r.T, r2(bn3_g), r2(bn3_b))

    ms = _agg_call(h, src_both, dst)
    out = _tc_call(_post_body, (N, C_OUT),
                   ms, cnt, h, W3l.T, r2(b3l), W3r.T, r2(bn4_g), r2(bn4_b),
                   W_post1.T, r2(b_post1), r2(bn5_g), r2(bn5_b),
                   W_post2.T, r2(b_post2))
    return out


# speedup vs baseline: 1.0168x; 1.0028x over previous
import jax
import jax.numpy as jnp
from jax import lax
from jax.experimental import pallas as pl
from jax.experimental.pallas import tpu as pltpu
from jax.experimental.pallas import tpu_sc as plsc

N = 10000
E = 320000
F_IN = 128
H = 256
HH = 128
C_OUT = 40
NT = 16
CH = 128
BI = 20
NBLK = 8
EPT = CH * BI * NBLK
EP = EPT * NT
NP = 10240
RPT = NP // NT
NCHT = EPT // CH
EPS = 1e-5


def _agg_body(h_hbm, src_hbm, dst_hbm, out_hbm,
              isrc0, isrc1, idst0, idst1, buf0, buf1, acc,
              g0, g1, g0b, g1b, p0, p1):
    c = lax.axis_index("c")
    s = lax.axis_index("s")

    zv = jnp.zeros((16,), jnp.float32)

    def _z(r, _):
        for k in range(HH // 16):
            buf0[r, pl.ds(k * 16, 16)] = zv
        return 0

    pltpu.async_copy(src_hbm.at[s, 0], isrc0, p0)
    pltpu.async_copy(dst_hbm.at[s, 0], idst0, p1)
    lax.fori_loop(0, CH, _z, 0)
    for k in range(RPT // CH):
        pltpu.async_copy(buf0, acc.at[pl.ds(s * RPT + k * CH, CH)], g1)
    for k in range(RPT // CH):
        pltpu.make_async_copy(buf0, acc.at[pl.ds(s * RPT + k * CH, CH)],
                              g1).wait()
    plsc.subcore_barrier()

    off = c * NP

    def _rebase(A):
        def _rb(i, _):
            r = i // (CH // 16)
            k = lax.rem(i, CH // 16) * 16
            A[r, pl.ds(k, 16)] = A[r, pl.ds(k, 16)] + off
            return 0
        lax.fori_loop(0, BI * (CH // 16), _rb, 0)

    def _do_block(A, B, A2, B2, i):
        if i > 0:
            pltpu.make_async_copy(src_hbm.at[s, i], A, p0).wait()
            pltpu.make_async_copy(dst_hbm.at[s, i], B, p1).wait()
            _rebase(A)
        if i + 1 < NBLK:
            pltpu.async_copy(src_hbm.at[s, i + 1], A2, p0)
            pltpu.async_copy(dst_hbm.at[s, i + 1], B2, p1)

        def _gather(j, bf, sa, sb):
            pltpu.async_copy(h_hbm.at[A.at[j, pl.ds(0, CH // 2)]],
                             bf.at[pl.ds(0, CH // 2)], sa)
            pltpu.async_copy(h_hbm.at[A.at[j, pl.ds(CH // 2, CH // 2)]],
                             bf.at[pl.ds(CH // 2, CH // 2)], sb)

        def _gwait(j, bf, sa, sb):
            pltpu.make_async_copy(h_hbm.at[A.at[j, pl.ds(0, CH // 2)]],
                                  bf.at[pl.ds(0, CH // 2)], sa).wait()
            pltpu.make_async_copy(h_hbm.at[A.at[j, pl.ds(CH // 2, CH // 2)]],
                                  bf.at[pl.ds(CH // 2, CH // 2)], sb).wait()

        _gather(0, buf0, g0, g0b)

        def _pair(t, _):
            j0 = 2 * t
            j1 = j0 + 1
            _gwait(j0, buf0, g0, g0b)
            _gather(j1, buf1, g1, g1b)
            pltpu.sync_copy(buf0, acc.at[B.at[j0]], add=True)
            _gwait(j1, buf1, g1, g1b)

            @pl.when(t + 1 < BI // 2)
            def _():
                _gather(j0 + 2, buf0, g0, g0b)

            pltpu.sync_copy(buf1, acc.at[B.at[j1]], add=True)
            return 0

        lax.fori_loop(0, BI // 2, _pair, 0)

    pltpu.make_async_copy(src_hbm.at[s, 0], isrc0, p0).wait()
    pltpu.make_async_copy(dst_hbm.at[s, 0], idst0, p1).wait()
    _rebase(isrc0)
    for i in range(NBLK):
        if i % 2 == 0:
            _do_block(isrc0, idst0, isrc1, idst1, i)
        else:
            _do_block(isrc1, idst1, isrc0, idst0, i)
    plsc.subcore_barrier()

    base = s * RPT
    for k in range(RPT // CH):
        pltpu.async_copy(acc.at[pl.ds(base + k * CH, CH)],
                         out_hbm.at[pl.ds(c * NP + base + k * CH, CH)], g0)
    for k in range(RPT // CH):
        pltpu.make_async_copy(acc.at[pl.ds(base + k * CH, CH)],
                              out_hbm.at[pl.ds(c * NP + base + k * CH, CH)],
                              g0).wait()


@jax.jit
def _agg_call(h_split, src_both, dst):
    mesh = plsc.VectorSubcoreMesh(core_axis_name="c", subcore_axis_name="s")
    return pl.kernel(
        _agg_body,
        out_type=jax.ShapeDtypeStruct((2 * NP, HH), jnp.float32),
        mesh=mesh,
        scratch_types=[
            pltpu.VMEM((BI, CH), jnp.int32),
            pltpu.VMEM((BI, CH), jnp.int32),
            pltpu.VMEM((BI, CH), jnp.int32),
            pltpu.VMEM((BI, CH), jnp.int32),
            pltpu.VMEM((CH, HH), jnp.float32),
            pltpu.VMEM((CH, HH), jnp.float32),
            pltpu.VMEM_SHARED((NP, HH), jnp.float32),
            pltpu.SemaphoreType.DMA,
            pltpu.SemaphoreType.DMA,
            pltpu.SemaphoreType.DMA,
            pltpu.SemaphoreType.DMA,
            pltpu.SemaphoreType.DMA,
            pltpu.SemaphoreType.DMA,
        ],
    )(h_split, src_both, dst)


def _cnt_body(dst_hbm, out_hbm, idx_d, ones_v, zb, cnt_acc):
    c = lax.axis_index("c")
    s = lax.axis_index("s")

    pltpu.sync_copy(dst_hbm.at[s], idx_d)

    ov = jnp.ones((16,), jnp.float32)
    for k in range(CH // 16):
        ones_v[pl.ds(k * 16, 16)] = ov

    zv = jnp.zeros((16,), jnp.float32)

    @pl.when(s == 0)
    def _():
        def _z(i, _):
            zb[pl.ds(i * 16, 16)] = zv
            return 0
        lax.fori_loop(0, NP // 16, _z, 0)
        pltpu.sync_copy(zb, cnt_acc)

    plsc.subcore_barrier()

    def _count(j, _):
        pltpu.sync_copy(ones_v, cnt_acc.at[idx_d.at[j]], add=True)
        return 0

    lax.fori_loop(0, NCHT, _count, 0)
    plsc.subcore_barrier()

    @pl.when(s == 0)
    def _():
        pltpu.sync_copy(cnt_acc.at[pl.ds(c * (N // 2), N // 2)],
                        zb.at[pl.ds(0, N // 2)])
        pltpu.sync_copy(zb.at[pl.ds(0, N // 2)],
                        out_hbm.at[pl.ds(c * (N // 2), N // 2)])


@jax.jit
def _cnt_call(dst):
    mesh = plsc.VectorSubcoreMesh(core_axis_name="c", subcore_axis_name="s")
    return pl.kernel(
        _cnt_body,
        out_type=jax.ShapeDtypeStruct((N,), jnp.float32),
        mesh=mesh,
        scratch_types=[
            pltpu.VMEM((NCHT, CH), jnp.int32),
            pltpu.VMEM((CH,), jnp.float32),
            pltpu.VMEM((NP,), jnp.float32),
            pltpu.VMEM_SHARED((NP,), jnp.float32),
        ],
    )(dst)


def _bn_relu(t, g, b):
    mu = jnp.mean(t, axis=0, keepdims=True)
    var = jnp.mean((t - mu) ** 2, axis=0, keepdims=True)
    y = (t - mu) * lax.rsqrt(var + EPS) * g + b
    return jnp.maximum(y, 0.0)


def _split_store(out_ref, y):
    out_ref[pl.ds(0, N), :] = y[:, :HH]
    out_ref[pl.ds(NP, N), :] = y[:, HH:]


def _dot16(a, w):
    return jnp.dot(a.astype(jnp.bfloat16), w.astype(jnp.bfloat16),
                   preferred_element_type=jnp.float32)


def _pre_body(x_ref, w_ref, b_ref, g_ref, bb_ref, out_ref):
    h = _dot16(x_ref[...], w_ref[...])
    y = _bn_relu(h + b_ref[...], g_ref[...], bb_ref[...])
    _split_store(out_ref, y)


def _sage_tail(ms_ref, cnt_ref, h_ref, wl_ref, bl_ref, wr_ref):
    inv = 1.0 / jnp.maximum(cnt_ref[...], 1.0)
    m0 = ms_ref[pl.ds(0, N), :] * inv
    m1 = ms_ref[pl.ds(NP, N), :] * inv
    h0 = h_ref[pl.ds(0, N), :]
    h1 = h_ref[pl.ds(NP, N), :]
    t = (_dot16(m0, wl_ref[pl.ds(0, HH), :])
         + _dot16(m1, wl_ref[pl.ds(HH, HH), :])
         + _dot16(h0, wr_ref[pl.ds(0, HH), :])
         + _dot16(h1, wr_ref[pl.ds(HH, HH), :]))
    return t + bl_ref[...]


def _mid_body(ms_ref, cnt_ref, h_ref, wl_ref, bl_ref, wr_ref, g_ref, bb_ref,
              out_ref):
    t = _sage_tail(ms_ref, cnt_ref, h_ref, wl_ref, bl_ref, wr_ref)
    y = _bn_relu(t, g_ref[...], bb_ref[...])
    _split_store(out_ref, y)


def _post_body(ms_ref, cnt_ref, h_ref, wl_ref, bl_ref, wr_ref, g4_ref, b4_ref,
               wp1_ref, bp1_ref, g5_ref, b5_ref, wp2_ref, bp2_ref, out_ref):
    t = _sage_tail(ms_ref, cnt_ref, h_ref, wl_ref, bl_ref, wr_ref)
    y = _bn_relu(t, g4_ref[...], b4_ref[...])
    u = _dot16(y, wp1_ref[...]) + bp1_ref[...]
    y5 = _bn_relu(u, g5_ref[...], b5_ref[...])
    z = _dot16(y5, wp2_ref[...]) + bp2_ref[...]
    zm = jnp.max(z, axis=1, keepdims=True)
    lse = jnp.log(jnp.sum(jnp.exp(z - zm), axis=1, keepdims=True)) + zm
    out_ref[...] = z - lse


def _tc_call(body, out_shape, *args):
    return pl.pallas_call(
        body, out_shape=jax.ShapeDtypeStruct(out_shape, jnp.float32))(*args)


def _pad_edges(edge_index):
    npad = EP - E
    pad_src = jnp.arange(npad, dtype=jnp.int32) % N
    pad_dst = N + jnp.arange(npad, dtype=jnp.int32) % (NP - N)
    src = jnp.concatenate([edge_index[0], pad_src])
    dst = jnp.concatenate([edge_index[1], pad_dst])
    return src.reshape(NT, NBLK, BI, CH), dst.reshape(NT, NBLK, BI, CH)


def kernel(x, edge_index, W_pre, b_pre, bn1_g, bn1_b, W1l, b1l, W1r,
           bn2_g, bn2_b, W2l, b2l, W2r, bn3_g, bn3_b, W3l, b3l, W3r,
           bn4_g, bn4_b, W_post1, b_post1, bn5_g, bn5_b, W_post2, b_post2):
    src_both, dst = _pad_edges(edge_index)

    cnt = _cnt_call(dst.reshape(NT, NCHT, CH)).reshape(N, 1)

    r2 = lambda v: v.reshape(1, -1)
    h = _tc_call(_pre_body, (2 * NP, HH),
                 x, W_pre.T, r2(b_pre), r2(bn1_g), r2(bn1_b))

    ms = _agg_call(h, src_both, dst)
    h = _tc_call(_mid_body, (2 * NP, HH),
                 ms, cnt, h, W1l.T, r2(b1l), W1r.T, r2(bn2_g), r2(bn2_b))

    ms = _agg_call(h, src_both, dst)
    h = _tc_call(_mid_body, (2 * NP, HH),
                 ms, cnt, h, W2l.T, r2(b2l), W2r.T, r2(bn3_g), r2(bn3_b))

    ms = _agg_call(h, src_both, dst)
    out = _tc_call(_post_body, (N, C_OUT),
                   ms, cnt, h, W3l.T, r2(b3l), W3r.T, r2(bn4_g), r2(bn4_b),
                   W_post1.T, r2(b_post1), r2(bn5_g), r2(bn5_b),
                   W_post2.T, r2(b_post2))
    return out
